# Initial kernel scaffold; baseline (speedup 1.0000x reference)
import functools
import jax
import jax.numpy as jnp
from jax import lax
from jax.experimental import pallas as pl
from jax.experimental.pallas import tpu as pltpu
from jax.experimental.pallas import tpu_sc as plsc

NC = 2
NS = 16
NW = NC * NS

CW = 100
GC = 8


def _make_kernel(B, S, V, D):
    CPS = S // CW
    CHUNKS = (B * S) // CW
    PER_W = CHUNKS // NW
    STEPS = PER_W // GC
    assert S % CW == 0 and CHUNKS % NW == 0 and PER_W % GC == 0
    HALF = D // 2

    mesh = plsc.VectorSubcoreMesh(core_axis_name="c", subcore_axis_name="s")

    @functools.partial(
        pl.kernel,
        mesh=mesh,
        out_type=jax.ShapeDtypeStruct((CHUNKS, CW, D), jnp.float32),
        scratch_types=[
            pltpu.VMEM((GC, CW), jnp.int32),
            pltpu.VMEM((GC, CW, D), jnp.float32),
            pltpu.VMEM((CPS, CW, D), jnp.float32),
            pltpu.SemaphoreType.DMA,
        ],
    )
    def k(idx_hbm, table_hbm, pos_hbm, out_hbm, idx_v, rows_v, pos_v, sem):
        wid = lax.axis_index("s") * NC + lax.axis_index("c")
        base = wid * PER_W
        pltpu.sync_copy(pos_hbm, pos_v)

        def step(g, carry):
            cbase = base + g * GC
            pltpu.sync_copy(idx_hbm.at[pl.ds(cbase, GC)], idx_v)
            copies = [
                pltpu.async_copy(table_hbm.at[idx_v.at[j]], rows_v.at[j], sem)
                for j in range(GC)
            ]
            for c in copies:
                c.wait()

            def add_row(r, inner):
                for j in range(GC):
                    p = j % CPS
                    plsc.addupdate(
                        rows_v.at[j, r, pl.ds(0, HALF)], pos_v[p, r, pl.ds(0, HALF)]
                    )
                    plsc.addupdate(
                        rows_v.at[j, r, pl.ds(HALF, HALF)],
                        pos_v[p, r, pl.ds(HALF, HALF)],
                    )
                return inner

            lax.fori_loop(0, CW, add_row, 0)
            pltpu.sync_copy(rows_v, out_hbm.at[pl.ds(cbase, GC)])
            return carry

        lax.fori_loop(0, STEPS, step, 0)

    return k


def kernel(inputs, word_table, pos_embed):
    B, S = inputs.shape
    V, D = word_table.shape
    idx = inputs.astype(jnp.int32).reshape((B * S) // CW, CW)
    pos = pos_embed.reshape(S // CW, CW, D)
    out = _make_kernel(B, S, V, D)(idx, word_table, pos)
    return out.reshape(B, S, D)


# speedup vs baseline: 3.1334x; 3.1334x over previous
import functools
import jax
import jax.numpy as jnp
from jax import lax
from jax.experimental import pallas as pl
from jax.experimental.pallas import tpu as pltpu
from jax.experimental.pallas import tpu_sc as plsc

NC = 2
NS = 16
NW = NC * NS

CW = 100
GC = 8


def _make_kernel(B, S, V, D):
    CPS = S // CW
    CHUNKS = (B * S) // CW
    PER_W = CHUNKS // NW
    STEPS = PER_W // GC
    assert S % CW == 0 and CHUNKS % NW == 0 and PER_W % GC == 0
    HALF = D // 2

    mesh = plsc.VectorSubcoreMesh(core_axis_name="c", subcore_axis_name="s")

    @functools.partial(
        pl.kernel,
        mesh=mesh,
        out_type=jax.ShapeDtypeStruct((CHUNKS, CW, D), jnp.float32),
        scratch_types=[
            pltpu.VMEM((GC, CW), jnp.int32),
            pltpu.VMEM((GC, CW, D), jnp.float32),
            pltpu.VMEM((CPS, CW, D), jnp.float32),
            pltpu.SemaphoreType.DMA,
        ],
        compiler_params=pltpu.CompilerParams(use_tc_tiling_on_sc=False),
    )
    def k(idx_hbm, table_hbm, pos_hbm, out_hbm, idx_v, rows_v, pos_v, sem):
        wid = lax.axis_index("s") * NC + lax.axis_index("c")
        base = wid * PER_W
        pltpu.sync_copy(pos_hbm, pos_v)

        def step(g, carry):
            cbase = base + g * GC
            pltpu.sync_copy(idx_hbm.at[pl.ds(cbase, GC)], idx_v)
            copies = [
                pltpu.async_copy(table_hbm.at[idx_v.at[j]], rows_v.at[j], sem)
                for j in range(GC)
            ]
            for c in copies:
                c.wait()

            def add_row(r, inner):
                for j in range(GC):
                    p = j % CPS
                    plsc.addupdate(
                        rows_v.at[j, r, pl.ds(0, HALF)], pos_v[p, r, pl.ds(0, HALF)]
                    )
                    plsc.addupdate(
                        rows_v.at[j, r, pl.ds(HALF, HALF)],
                        pos_v[p, r, pl.ds(HALF, HALF)],
                    )
                return inner

            lax.fori_loop(0, CW, add_row, 0)
            pltpu.sync_copy(rows_v, out_hbm.at[pl.ds(cbase, GC)])
            return carry

        lax.fori_loop(0, STEPS, step, 0)

    return k


def kernel(inputs, word_table, pos_embed):
    B, S = inputs.shape
    V, D = word_table.shape
    idx = inputs.astype(jnp.int32).reshape((B * S) // CW, CW)
    pos = pos_embed.reshape(S // CW, CW, D)
    out = _make_kernel(B, S, V, D)(idx, word_table, pos)
    return out.reshape(B, S, D)


# speedup vs baseline: 3.3191x; 1.0592x over previous
import functools
import jax
import jax.numpy as jnp
from jax import lax
from jax.experimental import pallas as pl
from jax.experimental.pallas import tpu as pltpu
from jax.experimental.pallas import tpu_sc as plsc

NC = 2
NS = 16
NW = NC * NS

CW = 100
GC = 8


def _make_kernel(B, S, V, D):
    CPS = S // CW
    CHUNKS = (B * S) // CW
    PER_W = CHUNKS // NW
    STEPS = PER_W // GC
    assert S % CW == 0 and CHUNKS % NW == 0 and PER_W % GC == 0
    assert STEPS % 2 == 0
    HALF = D // 2

    mesh = plsc.VectorSubcoreMesh(core_axis_name="c", subcore_axis_name="s")

    @functools.partial(
        pl.kernel,
        mesh=mesh,
        out_type=jax.ShapeDtypeStruct((CHUNKS, CW, D), jnp.float32),
        scratch_types=[
            pltpu.VMEM((2, GC, CW), jnp.int32),
            pltpu.VMEM((2, GC, CW, D), jnp.float32),
            pltpu.VMEM((CPS, CW, D), jnp.float32),
            pltpu.SemaphoreType.DMA,
            pltpu.SemaphoreType.DMA,
        ],
        compiler_params=pltpu.CompilerParams(use_tc_tiling_on_sc=False),
    )
    def k(idx_hbm, table_hbm, pos_hbm, out_hbm, idx_v, rows_v, pos_v, sem0, sem1):
        wid = lax.axis_index("s") * NC + lax.axis_index("c")
        base = wid * PER_W
        pltpu.sync_copy(pos_hbm, pos_v)
        sems = (sem0, sem1)

        def fire(s, b):
            idx_b = idx_v.at[b]
            pltpu.sync_copy(idx_hbm.at[pl.ds(base + s * GC, GC)], idx_b)
            for j in range(GC):
                pltpu.async_copy(table_hbm.at[idx_b.at[j]], rows_v.at[b, j], sems[b])

        def process(s, b):
            for j in range(GC):
                pltpu.make_async_copy(
                    table_hbm.at[idx_v.at[b, j]], rows_v.at[b, j], sems[b]
                ).wait()

            def add_row(r, inner):
                for j in range(GC):
                    p = j % CPS
                    plsc.addupdate(
                        rows_v.at[b, j, r, pl.ds(0, HALF)],
                        pos_v[p, r, pl.ds(0, HALF)],
                    )
                    plsc.addupdate(
                        rows_v.at[b, j, r, pl.ds(HALF, HALF)],
                        pos_v[p, r, pl.ds(HALF, HALF)],
                    )
                return inner

            lax.fori_loop(0, CW, add_row, 0)
            pltpu.sync_copy(rows_v.at[b], out_hbm.at[pl.ds(base + s * GC, GC)])

        fire(0, 0)

        def body(gg, carry):
            s0 = 2 * gg
            fire(s0 + 1, 1)
            process(s0, 0)

            @pl.when(s0 + 2 < STEPS)
            def _():
                fire(s0 + 2, 0)

            process(s0 + 1, 1)
            return carry

        lax.fori_loop(0, STEPS // 2, body, 0)

    return k


def kernel(inputs, word_table, pos_embed):
    B, S = inputs.shape
    V, D = word_table.shape
    idx = inputs.astype(jnp.int32).reshape((B * S) // CW, CW)
    pos = pos_embed.reshape(S // CW, CW, D)
    out = _make_kernel(B, S, V, D)(idx, word_table, pos)
    return out.reshape(B, S, D)


# speedup vs baseline: 4.9190x; 1.4821x over previous
import functools
import jax
import jax.numpy as jnp
from jax import lax
from jax.experimental import pallas as pl
from jax.experimental.pallas import tpu as pltpu
from jax.experimental.pallas import tpu_sc as plsc

NC = 2
NS = 16
NW = NC * NS

NB = 4
IW = 40


def _make_kernel(B, S, V, D):
    PER_W = B // NW
    STEPS = PER_W // NB
    GPS = S // IW
    assert B % NW == 0 and PER_W % NB == 0 and S % IW == 0 and STEPS % 2 == 0
    HALF = D // 2

    mesh = plsc.VectorSubcoreMesh(core_axis_name="c", subcore_axis_name="s")

    @functools.partial(
        pl.kernel,
        mesh=mesh,
        out_type=jax.ShapeDtypeStruct((B, S, D), jnp.float32),
        scratch_types=[
            pltpu.VMEM((2, NB, S), jnp.int32),
            pltpu.VMEM((2, NB, S, D), jnp.float32),
            pltpu.VMEM((S, D), jnp.float32),
            pltpu.SemaphoreType.DMA,
            pltpu.SemaphoreType.DMA,
        ],
        compiler_params=pltpu.CompilerParams(use_tc_tiling_on_sc=False),
    )
    def k(idx_hbm, table_hbm, pos_hbm, out_hbm, idx_v, rows_v, pos_v, sem0, sem1):
        wid = lax.axis_index("s") * NC + lax.axis_index("c")
        base = wid * PER_W
        pltpu.sync_copy(pos_hbm, pos_v)
        sems = (sem0, sem1)

        def fire(s, b):
            pltpu.sync_copy(idx_hbm.at[pl.ds(base + s * NB, NB)], idx_v.at[b])
            for jb in range(NB):
                for h in range(GPS):
                    pltpu.async_copy(
                        table_hbm.at[idx_v.at[b, jb, pl.ds(h * IW, IW)]],
                        rows_v.at[b, jb, pl.ds(h * IW, IW)],
                        sems[b],
                    )

        def process(s, b):
            for jb in range(NB):
                for h in range(GPS):
                    pltpu.make_async_copy(
                        table_hbm.at[idx_v.at[b, jb, pl.ds(h * IW, IW)]],
                        rows_v.at[b, jb, pl.ds(h * IW, IW)],
                        sems[b],
                    ).wait()

            def add_row(r, inner):
                for jb in range(NB):
                    plsc.addupdate(
                        rows_v.at[b, jb, r, pl.ds(0, HALF)],
                        pos_v[r, pl.ds(0, HALF)],
                    )
                    plsc.addupdate(
                        rows_v.at[b, jb, r, pl.ds(HALF, HALF)],
                        pos_v[r, pl.ds(HALF, HALF)],
                    )
                return inner

            lax.fori_loop(0, S, add_row, 0)
            pltpu.sync_copy(rows_v.at[b], out_hbm.at[pl.ds(base + s * NB, NB)])

        fire(0, 0)

        def body(gg, carry):
            s0 = 2 * gg
            fire(s0 + 1, 1)
            process(s0, 0)

            @pl.when(s0 + 2 < STEPS)
            def _():
                fire(s0 + 2, 0)

            process(s0 + 1, 1)
            return carry

        lax.fori_loop(0, STEPS // 2, body, 0)

    return k


def kernel(inputs, word_table, pos_embed):
    B, S = inputs.shape
    V, D = word_table.shape
    return _make_kernel(B, S, V, D)(
        inputs.astype(jnp.int32), word_table, pos_embed
    )
